# baseline (speedup 1.0000x reference)
import functools

import jax
import jax.numpy as jnp
from jax import lax
from jax.experimental import pallas as pl
from jax.experimental.pallas import tpu as pltpu
from jax.experimental.pallas import tpu_sc as plsc

B = 16384
F = 26
FIELD = 38462
OFFS = [f * FIELD for f in range(F)]
NC = 2
NS = 16
NW = NC * NS
BPW = B // NW
IPW = BPW * F
GROW = 128
NROW = IPW // GROW
CHUNKS = BPW // 16


def _body(xt_hbm, wf_hbm, bias_hbm, out_hbm,
          xv, idxv, vals, outv, bv, sem):
    wid = lax.axis_index("s") * NC + lax.axis_index("c")
    base = wid * BPW

    pltpu.sync_copy(xt_hbm.at[:, pl.ds(base, BPW)], xv)
    pltpu.sync_copy(bias_hbm, bv)

    def build(c, _):
        s = c * 16
        for f in range(F):
            idxv[pl.ds(f * BPW + s, 16)] = xv[f, pl.ds(s, 16)] + OFFS[f]
        return _
    lax.fori_loop(0, CHUNKS, build, None)

    def fire(j, _):
        pltpu.async_copy(
            wf_hbm.at[idxv.at[pl.ds(j * GROW, GROW)]],
            vals.at[pl.ds(j * GROW, GROW)],
            sem)
        return _
    lax.fori_loop(0, NROW, fire, None)
    pltpu.make_async_copy(wf_hbm.at[pl.ds(0, IPW)], vals, sem).wait()

    bias_v = bv[...]

    def reduce(c, _):
        s = c * 16
        acc = bias_v
        for f in range(F):
            acc = acc + vals[pl.ds(f * BPW + s, 16)]
        res = 1.0 / (1.0 + jnp.exp(-acc))
        outv[pl.ds(s, 16)] = res
        return _
    lax.fori_loop(0, CHUNKS, reduce, None)

    pltpu.sync_copy(outv, out_hbm.at[pl.ds(base, BPW)])


def kernel(x, W, bias):
    xt = x.astype(jnp.int32).T
    wf = W[:, 0]
    bias16 = jnp.broadcast_to(bias.astype(jnp.float32), (16,))

    mesh = plsc.VectorSubcoreMesh(core_axis_name="c", subcore_axis_name="s")
    run = functools.partial(
        pl.kernel,
        mesh=mesh,
        out_type=jax.ShapeDtypeStruct((B,), jnp.float32),
        scratch_types=[
            pltpu.VMEM((F, BPW), jnp.int32),
            pltpu.VMEM((IPW,), jnp.int32),
            pltpu.VMEM((IPW,), jnp.float32),
            pltpu.VMEM((BPW,), jnp.float32),
            pltpu.VMEM((16,), jnp.float32),
            pltpu.SemaphoreType.DMA,
        ],
    )(_body)
    return run(xt, wf, bias16)


# speedup vs baseline: 1.6804x; 1.6804x over previous
import functools

import jax
import jax.numpy as jnp
from jax import lax
from jax.experimental import pallas as pl
from jax.experimental.pallas import tpu as pltpu
from jax.experimental.pallas import tpu_sc as plsc

B = 16384
F = 26
FIELD = 38462
OFFS = [f * FIELD for f in range(F)]
NC = 2
NS = 16
NW = NC * NS
BPW = B // NW
IPW = BPW * F
GROW = 128
NROW = IPW // GROW
CHUNKS = BPW // 16


def _body(xt_hbm, wf_hbm, bias_hbm, out_hbm,
          xv, idxv, vals, outv, bv, sem):
    wid = lax.axis_index("s") * NC + lax.axis_index("c")
    base = wid * BPW

    pltpu.sync_copy(xt_hbm.at[:, pl.ds(base, BPW)], xv)
    pltpu.sync_copy(bias_hbm, bv)

    def build(c, _):
        s = c * 16
        for f in range(F):
            idxv[pl.ds(f * BPW + s, 16)] = xv[f, pl.ds(s, 16)] + OFFS[f]
        return _
    lax.fori_loop(0, CHUNKS, build, None)

    def fire(j, _):
        pltpu.async_copy(
            wf_hbm.at[idxv.at[pl.ds(j * GROW, GROW)]],
            vals.at[pl.ds(j * GROW, GROW)],
            sem)
        return _
    lax.fori_loop(0, NROW, fire, None)
    pltpu.make_async_copy(wf_hbm.at[pl.ds(0, IPW)], vals, sem).wait()

    bias_v = bv[...]

    def reduce(c, _):
        s = c * 16
        acc = bias_v
        for f in range(F):
            acc = acc + vals[pl.ds(f * BPW + s, 16)]
        res = 1.0 / (1.0 + jnp.exp(-acc))
        outv[pl.ds(s, 16)] = res
        return _
    lax.fori_loop(0, CHUNKS, reduce, None)

    pltpu.sync_copy(outv, out_hbm.at[pl.ds(base, BPW)])


def kernel(x, W, bias):
    xt = x.astype(jnp.int32).T
    pad = 1000448 - W.shape[0]
    wf = jnp.pad(W, ((0, pad), (0, 0))).reshape(-1)
    bias16 = jnp.broadcast_to(bias.astype(jnp.float32), (16,))

    mesh = plsc.VectorSubcoreMesh(core_axis_name="c", subcore_axis_name="s")
    run = functools.partial(
        pl.kernel,
        mesh=mesh,
        out_type=jax.ShapeDtypeStruct((B,), jnp.float32),
        scratch_types=[
            pltpu.VMEM((F, BPW), jnp.int32),
            pltpu.VMEM((IPW,), jnp.int32),
            pltpu.VMEM((IPW,), jnp.float32),
            pltpu.VMEM((BPW,), jnp.float32),
            pltpu.VMEM((16,), jnp.float32),
            pltpu.SemaphoreType.DMA,
        ],
    )(_body)
    return run(xt, wf, bias16)


# speedup vs baseline: 1.6906x; 1.0061x over previous
import functools

import jax
import jax.numpy as jnp
from jax import lax
from jax.experimental import pallas as pl
from jax.experimental.pallas import tpu as pltpu
from jax.experimental.pallas import tpu_sc as plsc

B = 16384
F = 26
FIELD = 38462
OFFS = [f * FIELD for f in range(F)]
NC = 2
NS = 16
NW = NC * NS
BPW = B // NW
IPW = BPW * F
GROW = 128
NROW = IPW // GROW
CHUNKS = BPW // 16


def _body(xt_hbm, wf_hbm, bias_hbm, out_hbm,
          xv, idxv, vals, outv, bv, sem):
    wid = lax.axis_index("s") * NC + lax.axis_index("c")
    base = wid * BPW

    pltpu.sync_copy(xt_hbm.at[:, pl.ds(base, BPW)], xv)
    pltpu.sync_copy(bias_hbm, bv)

    def build(c, _):
        s = c * 16
        for f in range(F):
            idxv[pl.ds(f * BPW + s, 16)] = xv[f, pl.ds(s, 16)] + OFFS[f]
        return _
    lax.fori_loop(0, CHUNKS, build, None)

    pltpu.async_copy(wf_hbm.at[idxv], vals, sem).wait()

    bias_v = bv[...]

    def reduce(c, _):
        s = c * 16
        acc = bias_v
        for f in range(F):
            acc = acc + vals[pl.ds(f * BPW + s, 16)]
        res = 1.0 / (1.0 + jnp.exp(-acc))
        outv[pl.ds(s, 16)] = res
        return _
    lax.fori_loop(0, CHUNKS, reduce, None)

    pltpu.sync_copy(outv, out_hbm.at[pl.ds(base, BPW)])


def kernel(x, W, bias):
    xt = x.astype(jnp.int32).T
    pad = 1000448 - W.shape[0]
    wf = jnp.pad(W, ((0, pad), (0, 0))).reshape(-1)
    bias16 = jnp.broadcast_to(bias.astype(jnp.float32), (16,))

    mesh = plsc.VectorSubcoreMesh(core_axis_name="c", subcore_axis_name="s")
    run = functools.partial(
        pl.kernel,
        mesh=mesh,
        out_type=jax.ShapeDtypeStruct((B,), jnp.float32),
        scratch_types=[
            pltpu.VMEM((F, BPW), jnp.int32),
            pltpu.VMEM((IPW,), jnp.int32),
            pltpu.VMEM((IPW,), jnp.float32),
            pltpu.VMEM((BPW,), jnp.float32),
            pltpu.VMEM((16,), jnp.float32),
            pltpu.SemaphoreType.DMA,
        ],
    )(_body)
    return run(xt, wf, bias16)
